# Initial kernel scaffold; baseline (speedup 1.0000x reference)
import jax
import jax.numpy as jnp
from jax.experimental import pallas as pl


def kernel(x):
    raise NotImplementedError("write your pallas kernel here")


# speedup vs baseline: 1.9431x; 1.9431x over previous
import functools

import jax
import jax.numpy as jnp
from jax import lax
from jax.experimental import pallas as pl
from jax.experimental.pallas import tpu as pltpu
from jax.experimental.pallas import tpu_sc as plsc

N_AGENTS = 1024
N_GOALS = 100000
D_FEAT = 64

LANE_T = 512
N_INNER = 14
GOAL_BLK = LANE_T * N_INNER
N_GRID = 14
N_GOALS_PAD = GOAL_BLK * N_GRID
PAD_VAL = 1.0e30

A_CH = 8
N_A_CH = N_AGENTS // A_CH


def _argmin_body(q_ref, kt_ref, bd_ref, bi_ref):
    gb = pl.program_id(0)

    @pl.when(gb == 0)
    def _init():
        bd_ref[...] = jnp.full((N_AGENTS, 1), jnp.inf, jnp.float32)
        bi_ref[...] = jnp.zeros((N_AGENTS, 1), jnp.int32)

    def body(a, carry):
        a8 = pl.ds(a * A_CH, A_CH)
        qb = q_ref[a8, :]
        for j in range(N_INNER):
            c0 = j * LANE_T
            accs = [None, None, None, None]
            for d in range(D_FEAT):
                krow = kt_ref[d:d + 1, c0:c0 + LANE_T]
                t = jnp.abs(qb[:, d:d + 1] - krow)
                k = d % 4
                accs[k] = t if accs[k] is None else accs[k] + t
            acc = (accs[0] + accs[1]) + (accs[2] + accs[3])
            m = jnp.min(acc, axis=1, keepdims=True)
            lane = lax.broadcasted_iota(jnp.int32, (A_CH, LANE_T), 1)
            il = jnp.min(jnp.where(acc <= m, lane, LANE_T),
                         axis=1, keepdims=True)
            gi = gb * GOAL_BLK + c0 + il
            bd = bd_ref[a8, :]
            better = m < bd
            bd_ref[a8, :] = jnp.where(better, m, bd)
            bi_ref[a8, :] = jnp.where(better, gi, bi_ref[a8, :])
        return carry

    lax.fori_loop(0, N_A_CH, body, 0)


def _l1_argmin(q, kt_pad):
    bd, bi = pl.pallas_call(
        _argmin_body,
        grid=(N_GRID,),
        in_specs=[
            pl.BlockSpec((N_AGENTS, D_FEAT), lambda g: (0, 0)),
            pl.BlockSpec((D_FEAT, GOAL_BLK), lambda g: (0, g)),
        ],
        out_specs=[
            pl.BlockSpec((N_AGENTS, 1), lambda g: (0, 0)),
            pl.BlockSpec((N_AGENTS, 1), lambda g: (0, 0)),
        ],
        out_shape=[
            jax.ShapeDtypeStruct((N_AGENTS, 1), jnp.float32),
            jax.ShapeDtypeStruct((N_AGENTS, 1), jnp.int32),
        ],
        compiler_params=pltpu.CompilerParams(
            dimension_semantics=("arbitrary",)),
    )(q, kt_pad)
    return bi[:, 0]


_SC_NC = 2
_SC_NS = 16
_SC_NW = _SC_NC * _SC_NS
_ROWS_W = N_AGENTS // _SC_NW


_GROW = 128


def _sc_gather_body(keys_hbm, idx_hbm, q_hbm, out_hbm, idx_v, rows_v, q_v,
                    dis_v, sem):
    wid = lax.axis_index("s") * _SC_NC + lax.axis_index("c")
    base = wid * _ROWS_W
    pltpu.sync_copy(idx_hbm.at[pl.ds(base, _ROWS_W)], idx_v)
    pltpu.async_copy(keys_hbm.at[idx_v], rows_v, sem).wait()
    pltpu.sync_copy(q_hbm.at[pl.ds(base, _ROWS_W)], q_v)
    for r in range(_ROWS_W):
        for c in range(D_FEAT // 16):
            sl = pl.ds(c * 16, 16)
            dis_v[r, sl] = rows_v[r, sl] - q_v[r, sl]
    pltpu.sync_copy(dis_v, out_hbm.at[pl.ds(base, _ROWS_W)])


_sc_gather = functools.partial(
    pl.kernel,
    out_type=jax.ShapeDtypeStruct((N_AGENTS, D_FEAT), jnp.float32),
    mesh=plsc.VectorSubcoreMesh(core_axis_name="c", subcore_axis_name="s"),
    scratch_types=[
        pltpu.VMEM((_ROWS_W,), jnp.int32),
        pltpu.VMEM((_ROWS_W, _GROW), jnp.float32),
        pltpu.VMEM((_ROWS_W, D_FEAT), jnp.float32),
        pltpu.VMEM((_ROWS_W, D_FEAT), jnp.float32),
        pltpu.SemaphoreType.DMA,
    ],
)(_sc_gather_body)


def kernel(x):
    q = x[:N_AGENTS, :D_FEAT]
    keys = x[N_AGENTS:, :D_FEAT]
    kt_pad = jnp.concatenate(
        [keys.T,
         jnp.full((D_FEAT, N_GOALS_PAD - N_GOALS), PAD_VAL, jnp.float32)],
        axis=1)
    keys_pad = jnp.pad(keys, ((0, 0), (0, _GROW - D_FEAT)))
    idx = _l1_argmin(q, kt_pad)
    dis = _sc_gather(keys_pad, idx, q)
    return idx, dis


# speedup vs baseline: 2.7447x; 1.4126x over previous
import functools

import jax
import jax.numpy as jnp
from jax import lax
from jax.experimental import pallas as pl
from jax.experimental.pallas import tpu as pltpu
from jax.experimental.pallas import tpu_sc as plsc

N_AGENTS = 1024
N_GOALS = 100000
D_FEAT = 64

LANE_T = 512
N_TILES = 196
N_GOALS_PAD = LANE_T * N_TILES
PAD_VAL = 1.0e30
A_CH = 16
N_A_CH = N_AGENTS // A_CH
_BIG_I = 2**30
N_UNROLL = 4
D_BLK = 16


_KPACK = 50000


def _argmin_body(q_ref, kt_ref, idx_ref, qsc_ref):
    qb = q_ref[...]
    for d in range(D_FEAT):
        qsc_ref[d * A_CH:(d + 1) * A_CH, :] = jnp.broadcast_to(
            qb[:, d:d + 1], (A_CH, 128))

    def body(jj, carry):
        vmin = list(carry[:4])
        vidx = list(carry[4:])
        for u in range(N_UNROLL):
            tile = jj * N_UNROLL + u
            c0 = tile * LANE_T
            acc = [[None, None] for _ in range(4)]
            for d in range(D_FEAT):
                qcol = qsc_ref[d * A_CH:(d + 1) * A_CH, :]
                par = d % 2
                for lt in range(4):
                    krow = kt_ref[d:d + 1, pl.ds(c0 + lt * 128, 128)]
                    t = jnp.abs(qcol - krow)
                    a = acc[lt]
                    a[par] = t if a[par] is None else a[par] + t
            for lt in range(4):
                s = acc[lt][0] + acc[lt][1]
                b = s < vmin[lt]
                vmin[lt] = jnp.where(b, s, vmin[lt])
                vidx[lt] = jnp.where(b, tile, vidx[lt])
        return (*vmin, *vidx)

    inf8 = jnp.full((A_CH, 128), jnp.inf, jnp.float32)
    zer8 = jnp.zeros((A_CH, 128), jnp.int32)
    out = lax.fori_loop(0, N_TILES // N_UNROLL, body,
                        (inf8, inf8, inf8, inf8, zer8, zer8, zer8, zer8))
    vmin = out[:4]
    vidx = out[4:]

    m4 = jnp.minimum(jnp.minimum(vmin[0], vmin[1]),
                     jnp.minimum(vmin[2], vmin[3]))
    m = jnp.min(m4, axis=1, keepdims=True)
    lane = lax.broadcasted_iota(jnp.int32, (A_CH, 128), 1)
    best = None
    for lt in range(4):
        cand = jnp.where(vmin[lt] <= m, vidx[lt] * LANE_T + lt * 128 + lane,
                         _BIG_I)
        best = cand if best is None else jnp.minimum(best, cand)
    idx_ref[...] = jnp.min(best, axis=1, keepdims=True)


def _l1_argmin(q, kt_pad):
    bi = pl.pallas_call(
        _argmin_body,
        grid=(N_A_CH,),
        in_specs=[
            pl.BlockSpec((A_CH, D_FEAT), lambda a: (a, 0)),
            pl.BlockSpec((D_FEAT, N_GOALS_PAD), lambda a: (0, 0)),
        ],
        out_specs=pl.BlockSpec((A_CH, 1), lambda a: (a, 0)),
        out_shape=jax.ShapeDtypeStruct((N_AGENTS, 1), jnp.int32),
        scratch_shapes=[pltpu.VMEM((D_FEAT * A_CH, 128), jnp.float32)],
        compiler_params=pltpu.CompilerParams(
            dimension_semantics=("arbitrary",)),
    )(q, kt_pad)
    return bi.reshape(N_AGENTS)


_SC_NC = 2
_SC_NS = 16
_SC_NW = _SC_NC * _SC_NS
_ROWS_W = N_AGENTS // _SC_NW


_GROW = 128


def _sc_gather_body(keys_hbm, idx_hbm, q_hbm, out_hbm, idx_v, rows_v, q_v,
                    dis_v, sem):
    wid = lax.axis_index("s") * _SC_NC + lax.axis_index("c")
    base = wid * _ROWS_W
    pltpu.sync_copy(idx_hbm.at[pl.ds(base, _ROWS_W)], idx_v)
    pltpu.async_copy(keys_hbm.at[idx_v], rows_v, sem).wait()
    pltpu.sync_copy(q_hbm.at[pl.ds(base, _ROWS_W)], q_v)
    for r in range(_ROWS_W):
        for c in range(D_FEAT // 16):
            sl = pl.ds(c * 16, 16)
            dis_v[r, sl] = rows_v[r, sl] - q_v[r, sl]
    pltpu.sync_copy(dis_v, out_hbm.at[pl.ds(base, _ROWS_W)])


_sc_gather = functools.partial(
    pl.kernel,
    out_type=jax.ShapeDtypeStruct((N_AGENTS, D_FEAT), jnp.float32),
    mesh=plsc.VectorSubcoreMesh(core_axis_name="c", subcore_axis_name="s"),
    scratch_types=[
        pltpu.VMEM((_ROWS_W,), jnp.int32),
        pltpu.VMEM((_ROWS_W, _GROW), jnp.float32),
        pltpu.VMEM((_ROWS_W, D_FEAT), jnp.float32),
        pltpu.VMEM((_ROWS_W, D_FEAT), jnp.float32),
        pltpu.SemaphoreType.DMA,
    ],
)(_sc_gather_body)


def kernel(x):
    q = x[:N_AGENTS, :D_FEAT]
    keys = x[N_AGENTS:, :D_FEAT]
    kt_pad = jnp.concatenate(
        [keys.T,
         jnp.full((D_FEAT, N_GOALS_PAD - N_GOALS), PAD_VAL, jnp.float32)],
        axis=1)
    idx = _l1_argmin(q, kt_pad)
    keys_pad = jnp.pad(keys, ((0, 0), (0, _GROW - D_FEAT)))
    dis = _sc_gather(keys_pad, idx, q)
    return idx, dis
